# baseline (speedup 1.0000x reference)
import jax
import jax.numpy as jnp
from jax.experimental import pallas as pl
from jax.experimental.pallas import tpu as pltpu

_BM = 2000
_KC = 2048


def _fused_kernel(x_ref, adj_ref, wt_ref, b_ref, out_ref, y_ref):
    m = pl.program_id(0)
    k = pl.program_id(1)
    n = x_ref.shape[0] * pl.num_programs(1)
    ky_rows = x_ref.shape[0]
    y_pad = y_ref.shape[0] - n

    @pl.when(m == 0)
    def _():
        @pl.when(k == 0)
        def _():
            y_ref[pl.ds(n, y_pad), :] = jnp.zeros(
                (y_pad, y_ref.shape[1]), jnp.bfloat16
            )

        xb = x_ref[...].astype(jnp.bfloat16)
        wb = wt_ref[...].astype(jnp.bfloat16)
        row = pl.multiple_of(k * ky_rows, ky_rows)
        y_ref[pl.ds(row, ky_rows), :] = jnp.dot(
            xb, wb, preferred_element_type=jnp.float32
        ).astype(jnp.bfloat16)

    @pl.when(m > 0)
    def _():
        ab = adj_ref[...].astype(jnp.bfloat16)
        n_k = pl.num_programs(1)
        valid = n - (n_k - 1) * _KC
        lane = jax.lax.broadcasted_iota(jnp.int32, ab.shape, 1)
        ab = jnp.where(
            jnp.logical_or(k < n_k - 1, lane < valid), ab, jnp.bfloat16(0)
        )
        yrow = pl.multiple_of(k * _KC, _KC)
        part = jnp.dot(
            ab,
            y_ref[pl.ds(yrow, _KC), :],
            preferred_element_type=jnp.float32,
        )

        @pl.when(k == 0)
        def _():
            out_ref[...] = part + b_ref[...]

        @pl.when(k > 0)
        def _():
            out_ref[...] = out_ref[...] + part


def kernel(x, adj, W, b):
    n, in_f = x.shape
    out_f = W.shape[0]
    wt = W.T
    b2 = b.reshape(1, out_f)
    n_k = (n + _KC - 1) // _KC
    ky_rows = n // n_k
    n_m = n // _BM
    y_rows = n_k * _KC

    out = pl.pallas_call(
        _fused_kernel,
        grid=(1 + n_m, n_k),
        in_specs=[
            pl.BlockSpec(
                (ky_rows, in_f),
                lambda m, k: (jnp.where(m == 0, k, n_k - 1), 0),
            ),
            pl.BlockSpec(
                (_BM, _KC),
                lambda m, k: (
                    jnp.maximum(m - 1, 0),
                    jnp.where(m == 0, 0, k),
                ),
            ),
            pl.BlockSpec((in_f, out_f), lambda m, k: (0, 0)),
            pl.BlockSpec((1, out_f), lambda m, k: (0, 0)),
        ],
        out_specs=pl.BlockSpec(
            (_BM, out_f), lambda m, k: (jnp.maximum(m - 1, 0), 0)
        ),
        out_shape=jax.ShapeDtypeStruct((n, out_f), jnp.float32),
        scratch_shapes=[pltpu.VMEM((y_rows, out_f), jnp.bfloat16)],
        compiler_params=pltpu.CompilerParams(
            dimension_semantics=("arbitrary", "arbitrary"),
            vmem_limit_bytes=60 * 1024 * 1024,
        ),
    )(x, adj, wt, b2)
    return out


# speedup vs baseline: 1.0059x; 1.0059x over previous
import jax
import jax.numpy as jnp
from jax.experimental import pallas as pl
from jax.experimental.pallas import tpu as pltpu

_BM = 400


def _fused_kernel(x_ref, adj_ref, wt_ref, b_ref, out_ref, y_ref):
    i = pl.program_id(0)

    @pl.when(i == 0)
    def _():
        xb = x_ref[...].astype(jnp.bfloat16)
        wb = wt_ref[...].astype(jnp.bfloat16)
        y_ref[...] = jnp.dot(
            xb, wb, preferred_element_type=jnp.float32
        ).astype(jnp.bfloat16)

    @pl.when(i > 0)
    def _():
        out_ref[...] = (
            jax.lax.dot_general(
                adj_ref[...],
                y_ref[...],
                (((1,), (0,)), ((), ())),
                preferred_element_type=jnp.float32,
                precision=jax.lax.Precision.DEFAULT,
            )
            + b_ref[...]
        )


def kernel(x, adj, W, b):
    n, in_f = x.shape
    out_f = W.shape[0]
    wt = W.T
    b2 = b.reshape(1, out_f)

    def _blk(i):
        return (jnp.maximum(i - 1, 0), 0)

    out = pl.pallas_call(
        _fused_kernel,
        grid=(1 + n // _BM,),
        in_specs=[
            pl.BlockSpec((n, in_f), lambda i: (0, 0)),
            pl.BlockSpec((_BM, n), _blk),
            pl.BlockSpec((in_f, out_f), lambda i: (0, 0)),
            pl.BlockSpec((1, out_f), lambda i: (0, 0)),
        ],
        out_specs=pl.BlockSpec((_BM, out_f), _blk),
        out_shape=jax.ShapeDtypeStruct((n, out_f), jnp.float32),
        scratch_shapes=[pltpu.VMEM((n, out_f), jnp.bfloat16)],
        compiler_params=pltpu.CompilerParams(
            dimension_semantics=("arbitrary",),
            vmem_limit_bytes=62 * 1024 * 1024,
        ),
    )(x, adj, wt, b2)
    return out
